# Initial kernel scaffold; baseline (speedup 1.0000x reference)
import jax
import jax.numpy as jnp
from jax.experimental import pallas as pl


def kernel(x, table):
    raise NotImplementedError("write your pallas kernel here")


# speedup vs baseline: 1.1036x; 1.1036x over previous
import functools

import jax
import jax.numpy as jnp
from jax import lax
from jax.experimental import pallas as pl
from jax.experimental.pallas import tpu as pltpu
from jax.experimental.pallas import tpu_sc as plsc

D = 32
NC, NS = 2, 16
NW = NC * NS
CHUNK = 1600


def _emb_body(idx_hbm, table_hbm, out_hbm, idx_v, rows_v, sem):
    b_per_w = idx_hbm.shape[0] // NW
    n_chunks = b_per_w // CHUNK
    wid = lax.axis_index("s") * NC + lax.axis_index("c")
    base = wid * b_per_w

    def step(g, carry):
        off = base + g * CHUNK
        pltpu.sync_copy(idx_hbm.at[pl.ds(off, CHUNK)], idx_v)
        pltpu.async_copy(table_hbm.at[idx_v], rows_v, sem).wait()
        pltpu.sync_copy(rows_v, out_hbm.at[pl.ds(off, CHUNK)])
        return carry

    lax.fori_loop(0, n_chunks, step, 0)


@functools.partial(jax.jit, static_argnums=(2,))
def _emb(idx, table, b_total):
    mesh = plsc.VectorSubcoreMesh(core_axis_name="c", subcore_axis_name="s")
    f = functools.partial(
        pl.kernel,
        mesh=mesh,
        out_type=jax.ShapeDtypeStruct((b_total, D), jnp.float32),
        scratch_types=[
            pltpu.VMEM((CHUNK,), jnp.int32),
            pltpu.VMEM((CHUNK, D), jnp.float32),
            pltpu.SemaphoreType.DMA,
        ],
        compiler_params=pltpu.CompilerParams(use_tc_tiling_on_sc=False),
    )(_emb_body)
    return f(idx, table)


def kernel(x, table):
    b, t = x.shape
    idx = x.reshape(b * t).astype(jnp.int32)
    out = _emb(idx, table, b * t)
    return out.reshape(b, t, D)


# speedup vs baseline: 1.1077x; 1.0036x over previous
import functools

import jax
import jax.numpy as jnp
from jax import lax
from jax.experimental import pallas as pl
from jax.experimental.pallas import tpu as pltpu
from jax.experimental.pallas import tpu_sc as plsc

D = 32
NC, NS = 2, 16
NW = NC * NS
CHUNK = 1600


def _emb_body(idx_hbm, table_hbm, out_hbm, idx_v, rows_v, gsem0, gsem1, osem0, osem1):
    b_per_w = idx_hbm.shape[0] // NW
    n = b_per_w // CHUNK
    wid = lax.axis_index("s") * NC + lax.axis_index("c")
    base = wid * b_per_w
    gsem = (gsem0, gsem1)
    osem = (osem0, osem1)

    def start_gather(g):
        p = g % 2
        pltpu.sync_copy(idx_hbm.at[pl.ds(base + g * CHUNK, CHUNK)], idx_v.at[p])
        return pltpu.async_copy(table_hbm.at[idx_v.at[p]], rows_v.at[p], gsem[p])

    gathers = {0: start_gather(0)}
    outs = {}
    for g in range(n):
        p = g % 2
        q = (g + 1) % 2
        if g + 1 < n:
            if g - 1 >= 0:
                outs[g - 1].wait()
            gathers[g + 1] = start_gather(g + 1)
        gathers[g].wait()
        outs[g] = pltpu.async_copy(
            rows_v.at[p], out_hbm.at[pl.ds(base + g * CHUNK, CHUNK)], osem[p])
    if n >= 2:
        outs[n - 2].wait()
    outs[n - 1].wait()


@functools.partial(jax.jit, static_argnums=(2,))
def _emb(idx, table, b_total):
    mesh = plsc.VectorSubcoreMesh(core_axis_name="c", subcore_axis_name="s")
    f = functools.partial(
        pl.kernel,
        mesh=mesh,
        out_type=jax.ShapeDtypeStruct((b_total, D), jnp.float32),
        scratch_types=[
            pltpu.VMEM((2, CHUNK), jnp.int32),
            pltpu.VMEM((2, CHUNK, D), jnp.float32),
            pltpu.SemaphoreType.DMA,
            pltpu.SemaphoreType.DMA,
            pltpu.SemaphoreType.DMA,
            pltpu.SemaphoreType.DMA,
        ],
        compiler_params=pltpu.CompilerParams(use_tc_tiling_on_sc=False),
    )(_emb_body)
    return f(idx, table)


def kernel(x, table):
    b, t = x.shape
    idx = x.reshape(b * t).astype(jnp.int32)
    out = _emb(idx, table, b * t)
    return out.reshape(b, t, D)


# speedup vs baseline: 1.1090x; 1.0012x over previous
import functools

import jax
import jax.numpy as jnp
from jax import lax
from jax.experimental import pallas as pl
from jax.experimental.pallas import tpu as pltpu
from jax.experimental.pallas import tpu_sc as plsc

D = 32
NC, NS = 2, 16
NW = NC * NS
CHUNK = 1600
NSTREAM = 4


def _emb_body(idx_hbm, table_hbm, out_hbm, idx_v, rows_v, gsem0, gsem1, osem0, osem1):
    b_per_w = idx_hbm.shape[0] // NW
    n = b_per_w // CHUNK
    wid = lax.axis_index("s") * NC + lax.axis_index("c")
    base = wid * b_per_w
    gsem = (gsem0, gsem1)
    osem = (osem0, osem1)

    sub = CHUNK // NSTREAM

    def start_gather(g):
        p = g % 2
        pltpu.sync_copy(idx_hbm.at[pl.ds(base + g * CHUNK, CHUNK)], idx_v.at[p])
        return [
            pltpu.async_copy(
                table_hbm.at[idx_v.at[p, pl.ds(s * sub, sub)]],
                rows_v.at[p, pl.ds(s * sub, sub)],
                gsem[p],
            )
            for s in range(NSTREAM)
        ]

    gathers = {0: start_gather(0)}
    outs = {}
    for g in range(n):
        p = g % 2
        if g + 1 < n:
            if g - 1 >= 0:
                outs[g - 1].wait()
            gathers[g + 1] = start_gather(g + 1)
        for c in gathers[g]:
            c.wait()
        outs[g] = pltpu.async_copy(
            rows_v.at[p], out_hbm.at[pl.ds(base + g * CHUNK, CHUNK)], osem[p])
    if n >= 2:
        outs[n - 2].wait()
    outs[n - 1].wait()


@functools.partial(jax.jit, static_argnums=(2,))
def _emb(idx, table, b_total):
    mesh = plsc.VectorSubcoreMesh(core_axis_name="c", subcore_axis_name="s")
    f = functools.partial(
        pl.kernel,
        mesh=mesh,
        out_type=jax.ShapeDtypeStruct((b_total, D), jnp.float32),
        scratch_types=[
            pltpu.VMEM((2, CHUNK), jnp.int32),
            pltpu.VMEM((2, CHUNK, D), jnp.float32),
            pltpu.SemaphoreType.DMA,
            pltpu.SemaphoreType.DMA,
            pltpu.SemaphoreType.DMA,
            pltpu.SemaphoreType.DMA,
        ],
        compiler_params=pltpu.CompilerParams(use_tc_tiling_on_sc=False),
    )(_emb_body)
    return f(idx, table)


def kernel(x, table):
    b, t = x.shape
    idx = x.reshape(b * t).astype(jnp.int32)
    out = _emb(idx, table, b * t)
    return out.reshape(b, t, D)
